# baseline (speedup 1.0000x reference)
import jax
import jax.numpy as jnp
import numpy as np
from jax import lax
from jax.experimental import pallas as pl
from jax.experimental.pallas import tpu as pltpu
from jax.experimental.pallas import tpu_sc as plsc

MIN_RES = 16
MAX_RES = 256
NUM_LOD = 10
FEAT = 8
N = 131072
_b = np.exp((np.log(MAX_RES) - np.log(MIN_RES)) / (NUM_LOD - 1))
LODS = [int(1 + np.floor(MIN_RES * _b ** l)) for l in range(NUM_LOD)]
CB_SIZE = 2 ** 19
MASK = CB_SIZE - 1
P2 = 265443567
P3 = 805459861

SIZES = [min(r ** 3, CB_SIZE) for r in LODS]
VP = [(s + 127) // 128 * 128 for s in SIZES]
NBLK = [v // 128 for v in VP]
N_DENSE = 6
DTOT = sum(VP[:N_DENSE])
HTOT = sum(VP[N_DENSE:])
OFFP = [sum(VP[:l]) for l in range(N_DENSE)]
OFFS = [sum(VP[N_DENSE:l]) for l in range(N_DENSE, NUM_LOD)]
DBLK = DTOT // 128
HBLK = HTOT // 128

NW = 32
PTS_W = N // NW
P = 256
NSUB = PTS_W // P
NV = P // 16
IDX_PER_DMA = 128
KRING = 16

f32 = jnp.float32
i32 = jnp.int32

_CPARAMS = pltpu.CompilerParams(
    needs_layout_passes=False, use_tc_tiling_on_sc=False)
_MESH = dict(core_axis_name="c", subcore_axis_name="s")


def _relayout_body(cb0, cb1, cb2, cb3, cb4, cb5, cb6, cb7, cb8, cb9,
                   bigp, bigs, inring, outring, insem, outsem):
    cbs = [cb0, cb1, cb2, cb3, cb4, cb5, cb6, cb7, cb8, cb9]
    w = lax.axis_index("s") * 2 + lax.axis_index("c")
    iota = lax.iota(i32, 16)
    fpat = iota & 7
    vpat = iota >> 3

    for l in range(NUM_LOD):
        src = cbs[l]
        dense = l < N_DENSE
        nin = 2 if dense else 1
        bl = NBLK[l]
        nblk = (bl - w + 31) >> 5
        kk = jnp.minimum(KRING, nblk)

        def fire_in(t):
            s = t & (KRING - 1)
            blk = w + t * 32
            pltpu.async_copy(src.at[blk],
                             inring.at[pl.ds(s * 16, 8)], insem)
            if nin == 2:
                pltpu.async_copy(src.at[blk + 1],
                                 inring.at[pl.ds(s * 16 + 8, 8)], insem)

        def prime(t, c):
            fire_in(t)
            return c

        lax.fori_loop(0, kk, prime, 0)

        def step(t, c):
            s = t & (KRING - 1)
            for q in range(nin):
                pltpu.make_async_copy(
                    src.at[0], inring.at[pl.ds(s * 16 + q * 8, 8)],
                    insem).wait()

            @pl.when(t >= KRING)
            def _():
                if dense:
                    pltpu.make_async_copy(bigp.at[0], outring.at[0],
                                          outsem).wait()
                else:
                    pltpu.make_async_copy(
                        bigs.at[0], outring.at[0, pl.ds(0, 1024)],
                        outsem).wait()

            rowv = fpat + s * 16
            if dense:
                for gb in range(8):
                    vals = [plsc.load_gather(
                        inring, [rowv, vpat + (gb * 16 + u)])
                        for u in range(16)]
                    for u in range(16):
                        g = gb * 16 + u
                        if g == 127:
                            continue
                        outring[s, pl.ds(g * 16, 16)] = vals[u]
                rv127 = fpat + (s * 16 + (vpat << 3))
                cv127 = (vpat + 127) & 127
                outring[s, pl.ds(127 * 16, 16)] = plsc.load_gather(
                    inring, [rv127, cv127])
                pltpu.async_copy(outring.at[s],
                                 bigp.at[OFFP[l] // 128 + w + t * 32],
                                 outsem)
            else:
                for gb in range(4):
                    vals = [plsc.load_gather(
                        inring, [rowv, vpat + 2 * (gb * 16 + u)])
                        for u in range(16)]
                    for u in range(16):
                        g = gb * 16 + u
                        outring[s, pl.ds(g * 16, 16)] = vals[u]
                pltpu.async_copy(
                    outring.at[s, pl.ds(0, 1024)],
                    bigs.at[OFFS[l - N_DENSE] // 128 + w + t * 32],
                    outsem)

            @pl.when(t + KRING < nblk)
            def _():
                fire_in(t + KRING)
            return c

        lax.fori_loop(0, nblk, step, 0)

        def drain(t, c):
            if dense:
                pltpu.make_async_copy(bigp.at[0], outring.at[0],
                                      outsem).wait()
            else:
                pltpu.make_async_copy(bigs.at[0],
                                      outring.at[0, pl.ds(0, 1024)],
                                      outsem).wait()
            return c

        lax.fori_loop(0, kk, drain, 0)


def _lookup_body(xh, yh, zh, bigp, bigs, out_h,
                 xv, yv, zv, ids_v, wts_v, rows_p, rows_s, acc_v,
                 sem0, sem1):
    wid = lax.axis_index("s") * 2 + lax.axis_index("c")
    base_pt = wid * PTS_W
    sems = [sem0, sem1]

    pltpu.sync_copy(xh.at[pl.ds(base_pt, PTS_W)], xv)
    pltpu.sync_copy(yh.at[pl.ds(base_pt, PTS_W)], yv)
    pltpu.sync_copy(zh.at[pl.ds(base_pt, PTS_W)], zv)

    iota = lax.iota(i32, 16)

    def run_subchunk(s, carry):
        sbase = s * P

        def make_pass_a(l, res, pp):
            scale = 0.5 * (res - 1)
            hi = np.float32(res - 1 - 1e-05)
            dense = l < N_DENSE
            res2 = res * res
            off = OFFP[l] if dense else OFFS[l - N_DENSE]

            def pass_a(j, c):
                o = sbase + j * 16
                x = xv[pl.ds(o, 16)]
                y = yv[pl.ds(o, 16)]
                z = zv[pl.ds(o, 16)]
                xf = (x + 1.0) * scale
                yf = (y + 1.0) * scale
                zf = (z + 1.0) * scale
                x1 = jnp.minimum(jnp.maximum(xf, 0.0), hi).astype(i32)
                y1 = jnp.minimum(jnp.maximum(yf, 0.0), hi).astype(i32)
                z1 = jnp.minimum(jnp.maximum(zf, 0.0), hi).astype(i32)
                x1f = x1.astype(f32)
                y1f = y1.astype(f32)
                z1f = z1.astype(f32)
                a1x = xf - x1f
                a1y = yf - y1f
                a1z = zf - z1f
                a0x = 1.0 - a1x
                a0y = 1.0 - a1y
                a0z = 1.0 - a1z
                if dense:
                    b = (z1 * res + y1) * res + x1 + off
                    ids = [b, b + res, b + res2, b + res2 + res]
                else:
                    hy0 = y1 * P2
                    hz0 = z1 * P3
                    hy1 = hy0 + P2
                    hz1 = hz0 + P3
                    x2 = x1 + 1
                    ids = [((x1 ^ hy0 ^ hz0) & MASK) + off,
                           ((x2 ^ hy0 ^ hz0) & MASK) + off,
                           ((x1 ^ hy1 ^ hz0) & MASK) + off,
                           ((x2 ^ hy1 ^ hz0) & MASK) + off,
                           ((x1 ^ hy0 ^ hz1) & MASK) + off,
                           ((x2 ^ hy0 ^ hz1) & MASK) + off,
                           ((x1 ^ hy1 ^ hz1) & MASK) + off,
                           ((x2 ^ hy1 ^ hz1) & MASK) + off]
                ws = [a0x * a0y * a0z, a1x * a0y * a0z,
                      a0x * a1y * a0z, a1x * a1y * a0z,
                      a0x * a0y * a1z, a1x * a0y * a1z,
                      a0x * a1y * a1z, a1x * a1y * a1z]
                jo = j * 16
                for c in range(len(ids)):
                    ids_v[pp, pl.ds(c * P + jo, 16)] = ids[c]
                for c in range(8):
                    wts_v[pp, pl.ds(c * P + jo, 16)] = ws[c]
                return c

            return pass_a

        def fire(l, pp):
            if l < N_DENSE:
                for d in range(4 * P // IDX_PER_DMA):
                    pltpu.async_copy(
                        bigp.at[ids_v.at[pp, pl.ds(d * IDX_PER_DMA,
                                                   IDX_PER_DMA)]],
                        rows_p.at[pp, pl.ds(d * IDX_PER_DMA,
                                            IDX_PER_DMA)],
                        sems[pp])
            else:
                for d in range(8 * P // IDX_PER_DMA):
                    pltpu.async_copy(
                        bigs.at[ids_v.at[pp, pl.ds(d * IDX_PER_DMA,
                                                   IDX_PER_DMA)]],
                        rows_s.at[pp, pl.ds(d * IDX_PER_DMA,
                                            IDX_PER_DMA)],
                        sems[pp])

        def drain(l, pp):
            if l < N_DENSE:
                for d in range(4 * P // IDX_PER_DMA):
                    pltpu.make_async_copy(
                        bigp.at[pl.ds(0, IDX_PER_DMA)],
                        rows_p.at[pp, pl.ds(d * IDX_PER_DMA,
                                            IDX_PER_DMA)],
                        sems[pp]).wait()
            else:
                for d in range(8 * P // IDX_PER_DMA):
                    pltpu.make_async_copy(
                        bigs.at[pl.ds(0, IDX_PER_DMA)],
                        rows_s.at[pp, pl.ds(d * IDX_PER_DMA,
                                            IDX_PER_DMA)],
                        sems[pp]).wait()

        def make_pass_b(l, pp):
            first = (l == 0)
            dense = l < N_DENSE

            def pass_b(j, c):
                jo = j * 16
                wv = [wts_v[pp, pl.ds(cc * P + jo, 16)] for cc in range(8)]
                rvec = iota + jo
                for f in range(8):
                    if dense:
                        gs = [plsc.load_gather(
                            rows_p.at[pp],
                            [rvec + (cc >> 1) * P,
                             jnp.full((16,), (cc & 1) * 8 + f, dtype=i32)])
                            for cc in range(8)]
                    else:
                        col = jnp.full((16,), f, dtype=i32)
                        gs = [plsc.load_gather(rows_s.at[pp],
                                               [rvec + cc * P, col])
                              for cc in range(8)]
                    ps = [wv[cc] * gs[cc] for cc in range(8)]
                    s01 = ps[0] + ps[1]
                    s23 = ps[2] + ps[3]
                    s45 = ps[4] + ps[5]
                    s67 = ps[6] + ps[7]
                    acc = (s01 + s23) + (s45 + s67)
                    acc = jnp.maximum(acc, 0.0)
                    if first:
                        acc_v[f, pl.ds(jo, 16)] = acc
                    else:
                        plsc.addupdate(acc_v.at[f, pl.ds(jo, 16)], acc)
                return c

            return pass_b

        lax.fori_loop(0, NV, make_pass_a(0, LODS[0], 0), 0)
        fire(0, 0)
        for l in range(1, NUM_LOD):
            pp = l & 1
            lax.fori_loop(0, NV, make_pass_a(l, LODS[l], pp), 0)
            fire(l, pp)
            drain(l - 1, 1 - pp)
            lax.fori_loop(0, NV, make_pass_b(l - 1, 1 - pp), 0)
        drain(NUM_LOD - 1, 1)
        lax.fori_loop(0, NV, make_pass_b(NUM_LOD - 1, 1), 0)

        for f in range(8):
            pltpu.sync_copy(acc_v.at[f],
                            out_h.at[f, pl.ds(base_pt + sbase, P)])
        return carry

    lax.fori_loop(0, NSUB, run_subchunk, 0)


def _relayout(*cb3s):
    kfn = pl.kernel(
        _relayout_body,
        out_type=(jax.ShapeDtypeStruct((DBLK, 2048), f32),
                  jax.ShapeDtypeStruct((HBLK, 1024), f32)),
        mesh=plsc.VectorSubcoreMesh(**_MESH),
        compiler_params=_CPARAMS,
        scratch_types=[
            pltpu.VMEM((KRING * 16, 128), f32),
            pltpu.VMEM((KRING, 2048), f32),
            pltpu.SemaphoreType.DMA,
            pltpu.SemaphoreType.DMA,
        ],
    )
    return kfn(*cb3s)


def _lookup(xs, ys, zs, bigp2, bigs2):
    kfn = pl.kernel(
        _lookup_body,
        out_type=jax.ShapeDtypeStruct((FEAT, N), f32),
        mesh=plsc.VectorSubcoreMesh(**_MESH),
        compiler_params=_CPARAMS,
        scratch_types=[
            pltpu.VMEM((PTS_W,), f32),
            pltpu.VMEM((PTS_W,), f32),
            pltpu.VMEM((PTS_W,), f32),
            pltpu.VMEM((2, 8 * P), i32),
            pltpu.VMEM((2, 8 * P), f32),
            pltpu.VMEM((2, 4 * P, 2 * FEAT), f32),
            pltpu.VMEM((2, 8 * P, FEAT), f32),
            pltpu.VMEM((FEAT, P), f32),
            pltpu.SemaphoreType.DMA,
            pltpu.SemaphoreType.DMA,
        ],
    )
    return kfn(xs, ys, zs, bigp2, bigs2)


def kernel(pts, codebook_0, codebook_1, codebook_2, codebook_3, codebook_4,
           codebook_5, codebook_6, codebook_7, codebook_8, codebook_9):
    cbs = [codebook_0, codebook_1, codebook_2, codebook_3, codebook_4,
           codebook_5, codebook_6, codebook_7, codebook_8, codebook_9]
    cb3s = []
    for l, cb in enumerate(cbs):
        v = cb.shape[0]
        vp = VP[l] + (128 if l < N_DENSE else 0)
        if vp != v:
            cb = jnp.pad(cb, ((0, vp - v), (0, 0)))
        cb3s.append(cb.reshape(vp // 128, 128, FEAT).transpose(0, 2, 1))
    bigp, bigs = _relayout(*cb3s)
    bigp2 = bigp.reshape(DTOT, 2 * FEAT)
    bigs2 = bigs.reshape(HTOT, FEAT)
    ptsT = pts.T
    outT = _lookup(ptsT[0], ptsT[1], ptsT[2], bigp2, bigs2)
    return outT.T


# speedup vs baseline: 1.0346x; 1.0346x over previous
import jax
import jax.numpy as jnp
import numpy as np
from jax import lax
from jax.experimental import pallas as pl
from jax.experimental.pallas import tpu as pltpu
from jax.experimental.pallas import tpu_sc as plsc

MIN_RES = 16
MAX_RES = 256
NUM_LOD = 10
FEAT = 8
N = 131072
_b = np.exp((np.log(MAX_RES) - np.log(MIN_RES)) / (NUM_LOD - 1))
LODS = [int(1 + np.floor(MIN_RES * _b ** l)) for l in range(NUM_LOD)]
CB_SIZE = 2 ** 19
MASK = CB_SIZE - 1
P2 = 265443567
P3 = 805459861

SIZES = [min(r ** 3, CB_SIZE) for r in LODS]
VP = [(s + 127) // 128 * 128 for s in SIZES]
OFF = [sum(VP[:l]) for l in range(NUM_LOD)]
TOT = sum(VP)
NBLK = [v // 128 for v in VP]
OFFB = [o // 128 for o in OFF]
NB = TOT // 128

NW = 32
PTS_W = N // NW
P = 512
NSUB = PTS_W // P
NV = P // 16
IDX_PER_DMA = 128
KRING = 16

f32 = jnp.float32
i32 = jnp.int32

_CPARAMS = pltpu.CompilerParams(
    needs_layout_passes=False, use_tc_tiling_on_sc=False)
_MESH = dict(core_axis_name="c", subcore_axis_name="s")


def _relayout_body(cb0, cb1, cb2, cb3, cb4, cb5, cb6, cb7, cb8, cb9,
                   big, inring, outring, insem, outsem):
    cbs = [cb0, cb1, cb2, cb3, cb4, cb5, cb6, cb7, cb8, cb9]
    w = lax.axis_index("s") * 2 + lax.axis_index("c")
    iota = lax.iota(i32, 16)
    fpat = iota & 7
    vpat = iota >> 3

    for l in range(NUM_LOD):
        src = cbs[l]
        bl = NBLK[l]
        nblk = (bl - w + 31) >> 5
        kk = jnp.minimum(KRING, nblk)

        def prime(t, c):
            s = t & (KRING - 1)
            pltpu.async_copy(src.at[w + t * 32],
                             inring.at[pl.ds(s * 8, 8)], insem)
            return c

        lax.fori_loop(0, kk, prime, 0)

        def step(t, c):
            s = t & (KRING - 1)
            pltpu.make_async_copy(src.at[0],
                                  inring.at[pl.ds(s * 8, 8)], insem).wait()

            @pl.when(t >= KRING)
            def _():
                pltpu.make_async_copy(src.at[0], outring.at[0],
                                      outsem).wait()

            rowv = fpat + s * 8
            for gb in range(4):
                vals = [plsc.load_gather(inring,
                                         [rowv, vpat + 2 * (gb * 16 + u)])
                        for u in range(16)]
                for u in range(16):
                    g = gb * 16 + u
                    outring[s, pl.ds(g * 16, 16)] = vals[u]

            pltpu.async_copy(outring.at[s], big.at[OFFB[l] + w + t * 32],
                             outsem)

            @pl.when(t + KRING < nblk)
            def _():
                s2 = (t + KRING) & (KRING - 1)
                pltpu.async_copy(src.at[w + (t + KRING) * 32],
                                 inring.at[pl.ds(s2 * 8, 8)], insem)
            return c

        lax.fori_loop(0, nblk, step, 0)

        def drain(t, c):
            pltpu.make_async_copy(src.at[0], outring.at[0], outsem).wait()
            return c

        lax.fori_loop(0, kk, drain, 0)


def _lookup_body(xh, yh, zh, big, out_h,
                 xv, yv, zv, ids_v, wts_v, rows_v, acc_v, sem0, sem1):
    wid = lax.axis_index("s") * 2 + lax.axis_index("c")
    base_pt = wid * PTS_W
    sems = [sem0, sem1]

    pltpu.sync_copy(xh.at[pl.ds(base_pt, PTS_W)], xv)
    pltpu.sync_copy(yh.at[pl.ds(base_pt, PTS_W)], yv)
    pltpu.sync_copy(zh.at[pl.ds(base_pt, PTS_W)], zv)

    iota = lax.iota(i32, 16)

    def run_subchunk(s, carry):
        sbase = s * P

        def make_pass_a(l, res, pp):
            scale = 0.5 * (res - 1)
            hi = np.float32(res - 1 - 1e-05)
            dense = res ** 3 <= CB_SIZE
            res2 = res * res
            off = OFF[l]

            def pass_a(j, c):
                o = sbase + j * 16
                x = xv[pl.ds(o, 16)]
                y = yv[pl.ds(o, 16)]
                z = zv[pl.ds(o, 16)]
                xf = (x + 1.0) * scale
                yf = (y + 1.0) * scale
                zf = (z + 1.0) * scale
                x1 = jnp.minimum(jnp.maximum(xf, 0.0), hi).astype(i32)
                y1 = jnp.minimum(jnp.maximum(yf, 0.0), hi).astype(i32)
                z1 = jnp.minimum(jnp.maximum(zf, 0.0), hi).astype(i32)
                x1f = x1.astype(f32)
                y1f = y1.astype(f32)
                z1f = z1.astype(f32)
                a1x = xf - x1f
                a1y = yf - y1f
                a1z = zf - z1f
                a0x = 1.0 - a1x
                a0y = 1.0 - a1y
                a0z = 1.0 - a1z
                if dense:
                    b = (z1 * res + y1) * res + x1 + off
                    ids = [b, b + 1, b + res, b + res + 1,
                           b + res2, b + res2 + 1,
                           b + res2 + res, b + res2 + res + 1]
                else:
                    hy0 = y1 * P2
                    hz0 = z1 * P3
                    hy1 = hy0 + P2
                    hz1 = hz0 + P3
                    x2 = x1 + 1
                    ids = [((x1 ^ hy0 ^ hz0) & MASK) + off,
                           ((x2 ^ hy0 ^ hz0) & MASK) + off,
                           ((x1 ^ hy1 ^ hz0) & MASK) + off,
                           ((x2 ^ hy1 ^ hz0) & MASK) + off,
                           ((x1 ^ hy0 ^ hz1) & MASK) + off,
                           ((x2 ^ hy0 ^ hz1) & MASK) + off,
                           ((x1 ^ hy1 ^ hz1) & MASK) + off,
                           ((x2 ^ hy1 ^ hz1) & MASK) + off]
                ws = [a0x * a0y * a0z, a1x * a0y * a0z,
                      a0x * a1y * a0z, a1x * a1y * a0z,
                      a0x * a0y * a1z, a1x * a0y * a1z,
                      a0x * a1y * a1z, a1x * a1y * a1z]
                jo = j * 16
                for c in range(8):
                    ids_v[pp, pl.ds(c * P + jo, 16)] = ids[c]
                    wts_v[pp, pl.ds(c * P + jo, 16)] = ws[c]
                return c

            return pass_a

        def fire(pp):
            pltpu.async_copy(big.at[ids_v.at[pp]], rows_v.at[pp],
                             sems[pp])

        def drain(pp):
            pltpu.make_async_copy(big.at[pl.ds(0, 8 * P)],
                                  rows_v.at[pp], sems[pp]).wait()

        def make_pass_b(l, pp):
            first = (l == 0)

            def pass_b(j, c):
                jo = j * 16
                wv = [wts_v[pp, pl.ds(cc * P + jo, 16)] for cc in range(8)]
                rvec = iota + jo
                for f in range(8):
                    col = jnp.full((16,), f, dtype=i32)
                    gs = [plsc.load_gather(rows_v.at[pp],
                                           [rvec + cc * P, col])
                          for cc in range(8)]
                    ps = [wv[cc] * gs[cc] for cc in range(8)]
                    s01 = ps[0] + ps[1]
                    s23 = ps[2] + ps[3]
                    s45 = ps[4] + ps[5]
                    s67 = ps[6] + ps[7]
                    acc = (s01 + s23) + (s45 + s67)
                    acc = jnp.maximum(acc, 0.0)
                    if first:
                        acc_v[f, pl.ds(jo, 16)] = acc
                    else:
                        plsc.addupdate(acc_v.at[f, pl.ds(jo, 16)], acc)
                return c

            return pass_b

        lax.fori_loop(0, NV, make_pass_a(0, LODS[0], 0), 0)
        fire(0)
        for l in range(1, NUM_LOD):
            pp = l & 1
            lax.fori_loop(0, NV, make_pass_a(l, LODS[l], pp), 0)
            fire(pp)
            drain(1 - pp)
            lax.fori_loop(0, NV, make_pass_b(l - 1, 1 - pp), 0)
        drain(1)
        lax.fori_loop(0, NV, make_pass_b(NUM_LOD - 1, 1), 0)

        for f in range(8):
            pltpu.sync_copy(acc_v.at[f],
                            out_h.at[f, pl.ds(base_pt + sbase, P)])
        return carry

    lax.fori_loop(0, NSUB, run_subchunk, 0)


def _relayout(*cb3s):
    kfn = pl.kernel(
        _relayout_body,
        out_type=jax.ShapeDtypeStruct((NB, 1024), f32),
        mesh=plsc.VectorSubcoreMesh(**_MESH),
        compiler_params=_CPARAMS,
        scratch_types=[
            pltpu.VMEM((KRING * 8, 128), f32),
            pltpu.VMEM((KRING, 1024), f32),
            pltpu.SemaphoreType.DMA,
            pltpu.SemaphoreType.DMA,
        ],
    )
    return kfn(*cb3s)


def _lookup(xs, ys, zs, big2):
    kfn = pl.kernel(
        _lookup_body,
        out_type=jax.ShapeDtypeStruct((FEAT, N), f32),
        mesh=plsc.VectorSubcoreMesh(**_MESH),
        compiler_params=_CPARAMS,
        scratch_types=[
            pltpu.VMEM((PTS_W,), f32),
            pltpu.VMEM((PTS_W,), f32),
            pltpu.VMEM((PTS_W,), f32),
            pltpu.VMEM((2, 8 * P), i32),
            pltpu.VMEM((2, 8 * P), f32),
            pltpu.VMEM((2, 8 * P, FEAT), f32),
            pltpu.VMEM((FEAT, P), f32),
            pltpu.SemaphoreType.DMA,
            pltpu.SemaphoreType.DMA,
        ],
    )
    return kfn(xs, ys, zs, big2)


def kernel(pts, codebook_0, codebook_1, codebook_2, codebook_3, codebook_4,
           codebook_5, codebook_6, codebook_7, codebook_8, codebook_9):
    cbs = [codebook_0, codebook_1, codebook_2, codebook_3, codebook_4,
           codebook_5, codebook_6, codebook_7, codebook_8, codebook_9]
    cb3s = []
    for l, cb in enumerate(cbs):
        v = cb.shape[0]
        if VP[l] != v:
            cb = jnp.pad(cb, ((0, VP[l] - v), (0, 0)))
        cb3s.append(cb.reshape(VP[l] // 128, 128, FEAT).transpose(0, 2, 1))
    big = _relayout(*cb3s)
    big2 = big.reshape(TOT, FEAT)
    ptsT = pts.T
    outT = _lookup(ptsT[0], ptsT[1], ptsT[2], big2)
    return outT.T


# speedup vs baseline: 1.0544x; 1.0191x over previous
import jax
import jax.numpy as jnp
import numpy as np
from jax import lax
from jax.experimental import pallas as pl
from jax.experimental.pallas import tpu as pltpu
from jax.experimental.pallas import tpu_sc as plsc

MIN_RES = 16
MAX_RES = 256
NUM_LOD = 10
FEAT = 8
N = 131072
_b = np.exp((np.log(MAX_RES) - np.log(MIN_RES)) / (NUM_LOD - 1))
LODS = [int(1 + np.floor(MIN_RES * _b ** l)) for l in range(NUM_LOD)]
CB_SIZE = 2 ** 19
MASK = CB_SIZE - 1
P2 = 265443567
P3 = 805459861

SIZES = [min(r ** 3, CB_SIZE) for r in LODS]
VP = [(s + 127) // 128 * 128 for s in SIZES]
OFF = [sum(VP[:l]) for l in range(NUM_LOD)]
TOT = sum(VP)
NBLK = [v // 128 for v in VP]
OFFB = [o // 128 for o in OFF]
NB = TOT // 128

NW = 32
PTS_W = N // NW
P = 256
NSUB = PTS_W // P
NV = P // 16
IDX_PER_DMA = 128
KRING = 16

f32 = jnp.float32
i32 = jnp.int32

_CPARAMS = pltpu.CompilerParams(
    needs_layout_passes=False, use_tc_tiling_on_sc=False)
_MESH = dict(core_axis_name="c", subcore_axis_name="s")


def _relayout_body(cb0, cb1, cb2, cb3, cb4, cb5, cb6, cb7, cb8, cb9,
                   big, inring, outring, insem, outsem):
    cbs = [cb0, cb1, cb2, cb3, cb4, cb5, cb6, cb7, cb8, cb9]
    w = lax.axis_index("s") * 2 + lax.axis_index("c")
    iota = lax.iota(i32, 16)
    fpat = iota & 7
    vpat = iota >> 3

    for l in range(NUM_LOD):
        src = cbs[l]
        bl = NBLK[l]
        nblk = (bl - w + 31) >> 5
        kk = jnp.minimum(KRING, nblk)

        def prime(t, c):
            s = t & (KRING - 1)
            pltpu.async_copy(src.at[w + t * 32],
                             inring.at[pl.ds(s * 8, 8)], insem)
            return c

        lax.fori_loop(0, kk, prime, 0)

        def step(t, c):
            s = t & (KRING - 1)
            pltpu.make_async_copy(src.at[0],
                                  inring.at[pl.ds(s * 8, 8)], insem).wait()

            @pl.when(t >= KRING)
            def _():
                pltpu.make_async_copy(src.at[0], outring.at[0],
                                      outsem).wait()

            rowv = fpat + s * 8
            for gb in range(4):
                vals = [plsc.load_gather(inring,
                                         [rowv, vpat + 2 * (gb * 16 + u)])
                        for u in range(16)]
                for u in range(16):
                    g = gb * 16 + u
                    outring[s, pl.ds(g * 16, 16)] = vals[u]

            pltpu.async_copy(outring.at[s], big.at[OFFB[l] + w + t * 32],
                             outsem)

            @pl.when(t + KRING < nblk)
            def _():
                s2 = (t + KRING) & (KRING - 1)
                pltpu.async_copy(src.at[w + (t + KRING) * 32],
                                 inring.at[pl.ds(s2 * 8, 8)], insem)
            return c

        lax.fori_loop(0, nblk, step, 0)

        def drain(t, c):
            pltpu.make_async_copy(src.at[0], outring.at[0], outsem).wait()
            return c

        lax.fori_loop(0, kk, drain, 0)


def _lookup_body(xh, yh, zh, big, out_h,
                 xv, yv, zv, ids_v, wts_v, rows_v, acc_v,
                 sem0, sem1, sem2):
    wid = lax.axis_index("s") * 2 + lax.axis_index("c")
    base_pt = wid * PTS_W
    sems = [sem0, sem1, sem2]

    pltpu.sync_copy(xh.at[pl.ds(base_pt, PTS_W)], xv)
    pltpu.sync_copy(yh.at[pl.ds(base_pt, PTS_W)], yv)
    pltpu.sync_copy(zh.at[pl.ds(base_pt, PTS_W)], zv)

    iota = lax.iota(i32, 16)

    def run_subchunk(s, carry):
        sbase = s * P

        def make_pass_a(l, res, pp):
            scale = 0.5 * (res - 1)
            hi = np.float32(res - 1 - 1e-05)
            dense = res ** 3 <= CB_SIZE
            res2 = res * res
            off = OFF[l]

            def pass_a(j, c):
                o = sbase + j * 16
                x = xv[pl.ds(o, 16)]
                y = yv[pl.ds(o, 16)]
                z = zv[pl.ds(o, 16)]
                xf = (x + 1.0) * scale
                yf = (y + 1.0) * scale
                zf = (z + 1.0) * scale
                x1 = jnp.minimum(jnp.maximum(xf, 0.0), hi).astype(i32)
                y1 = jnp.minimum(jnp.maximum(yf, 0.0), hi).astype(i32)
                z1 = jnp.minimum(jnp.maximum(zf, 0.0), hi).astype(i32)
                x1f = x1.astype(f32)
                y1f = y1.astype(f32)
                z1f = z1.astype(f32)
                a1x = xf - x1f
                a1y = yf - y1f
                a1z = zf - z1f
                a0x = 1.0 - a1x
                a0y = 1.0 - a1y
                a0z = 1.0 - a1z
                if dense:
                    b = (z1 * res + y1) * res + x1 + off
                    ids = [b, b + 1, b + res, b + res + 1,
                           b + res2, b + res2 + 1,
                           b + res2 + res, b + res2 + res + 1]
                else:
                    hy0 = y1 * P2
                    hz0 = z1 * P3
                    hy1 = hy0 + P2
                    hz1 = hz0 + P3
                    x2 = x1 + 1
                    ids = [((x1 ^ hy0 ^ hz0) & MASK) + off,
                           ((x2 ^ hy0 ^ hz0) & MASK) + off,
                           ((x1 ^ hy1 ^ hz0) & MASK) + off,
                           ((x2 ^ hy1 ^ hz0) & MASK) + off,
                           ((x1 ^ hy0 ^ hz1) & MASK) + off,
                           ((x2 ^ hy0 ^ hz1) & MASK) + off,
                           ((x1 ^ hy1 ^ hz1) & MASK) + off,
                           ((x2 ^ hy1 ^ hz1) & MASK) + off]
                ws = [a0x * a0y * a0z, a1x * a0y * a0z,
                      a0x * a1y * a0z, a1x * a1y * a0z,
                      a0x * a0y * a1z, a1x * a0y * a1z,
                      a0x * a1y * a1z, a1x * a1y * a1z]
                jo = j * 16
                for c in range(8):
                    ids_v[pp, pl.ds(c * P + jo, 16)] = ids[c]
                    wts_v[pp, pl.ds(c * P + jo, 16)] = ws[c]
                return c

            return pass_a

        def fire(pp):
            pltpu.async_copy(big.at[ids_v.at[pp]], rows_v.at[pp],
                             sems[pp])

        def drain(pp):
            pltpu.make_async_copy(big.at[pl.ds(0, 8 * P)],
                                  rows_v.at[pp], sems[pp]).wait()

        def make_pass_b(l, pp):
            first = (l == 0)

            def pass_b(j, c):
                jo = j * 16
                wv = [wts_v[pp, pl.ds(cc * P + jo, 16)] for cc in range(8)]
                rvec = iota + jo
                for f in range(8):
                    col = jnp.full((16,), f, dtype=i32)
                    gs = [plsc.load_gather(rows_v.at[pp],
                                           [rvec + cc * P, col])
                          for cc in range(8)]
                    ps = [wv[cc] * gs[cc] for cc in range(8)]
                    s01 = ps[0] + ps[1]
                    s23 = ps[2] + ps[3]
                    s45 = ps[4] + ps[5]
                    s67 = ps[6] + ps[7]
                    acc = (s01 + s23) + (s45 + s67)
                    acc = jnp.maximum(acc, 0.0)
                    if first:
                        acc_v[f, pl.ds(jo, 16)] = acc
                    else:
                        plsc.addupdate(acc_v.at[f, pl.ds(jo, 16)], acc)
                return c

            return pass_b

        for l in range(2):
            lax.fori_loop(0, NV, make_pass_a(l, LODS[l], l % 3), 0)
            fire(l % 3)
        for l in range(2, NUM_LOD):
            pp = (l - 2) % 3
            drain(pp)
            lax.fori_loop(0, NV, make_pass_b(l - 2, pp), 0)
            lax.fori_loop(0, NV, make_pass_a(l, LODS[l], l % 3), 0)
            fire(l % 3)
        for l in range(NUM_LOD - 2, NUM_LOD):
            pp = l % 3
            drain(pp)
            lax.fori_loop(0, NV, make_pass_b(l, pp), 0)

        for f in range(8):
            pltpu.sync_copy(acc_v.at[f],
                            out_h.at[f, pl.ds(base_pt + sbase, P)])
        return carry

    lax.fori_loop(0, NSUB, run_subchunk, 0)


def _relayout(*cb3s):
    kfn = pl.kernel(
        _relayout_body,
        out_type=jax.ShapeDtypeStruct((NB, 1024), f32),
        mesh=plsc.VectorSubcoreMesh(**_MESH),
        compiler_params=_CPARAMS,
        scratch_types=[
            pltpu.VMEM((KRING * 8, 128), f32),
            pltpu.VMEM((KRING, 1024), f32),
            pltpu.SemaphoreType.DMA,
            pltpu.SemaphoreType.DMA,
        ],
    )
    return kfn(*cb3s)


def _lookup(xs, ys, zs, big2):
    kfn = pl.kernel(
        _lookup_body,
        out_type=jax.ShapeDtypeStruct((FEAT, N), f32),
        mesh=plsc.VectorSubcoreMesh(**_MESH),
        compiler_params=_CPARAMS,
        scratch_types=[
            pltpu.VMEM((PTS_W,), f32),
            pltpu.VMEM((PTS_W,), f32),
            pltpu.VMEM((PTS_W,), f32),
            pltpu.VMEM((3, 8 * P), i32),
            pltpu.VMEM((3, 8 * P), f32),
            pltpu.VMEM((3, 8 * P, FEAT), f32),
            pltpu.VMEM((FEAT, P), f32),
            pltpu.SemaphoreType.DMA,
            pltpu.SemaphoreType.DMA,
            pltpu.SemaphoreType.DMA,
        ],
    )
    return kfn(xs, ys, zs, big2)


def kernel(pts, codebook_0, codebook_1, codebook_2, codebook_3, codebook_4,
           codebook_5, codebook_6, codebook_7, codebook_8, codebook_9):
    cbs = [codebook_0, codebook_1, codebook_2, codebook_3, codebook_4,
           codebook_5, codebook_6, codebook_7, codebook_8, codebook_9]
    cb3s = []
    for l, cb in enumerate(cbs):
        v = cb.shape[0]
        if VP[l] != v:
            cb = jnp.pad(cb, ((0, VP[l] - v), (0, 0)))
        cb3s.append(cb.reshape(VP[l] // 128, 128, FEAT).transpose(0, 2, 1))
    big = _relayout(*cb3s)
    big2 = big.reshape(TOT, FEAT)
    ptsT = pts.T
    outT = _lookup(ptsT[0], ptsT[1], ptsT[2], big2)
    return outT.T
